# Initial kernel scaffold; baseline (speedup 1.0000x reference)
import jax
import jax.numpy as jnp
from jax.experimental import pallas as pl


def kernel(x, alpha, edge_index, batch):
    raise NotImplementedError("write your pallas kernel here")


# speedup vs baseline: 42.1283x; 42.1283x over previous
import functools

import jax
import jax.numpy as jnp
from jax import lax
from jax.experimental import pallas as pl
from jax.experimental.pallas import tpu as pltpu
from jax.experimental.pallas import tpu_sc as plsc

N = 10000
E = 320000
G = 100
STEPS = 3
EPS = 0.01
CLAMP_STEP = 0.1

B = 256
NPAD = 10240
T = NPAD // B
NTILES = 32
EPT = E // NTILES
LANES = 16
PAD_SENT = 2 ** 30


def _prep_body(batch_ref, sqrtcnt_ref, invcnt_ref):
    b = batch_ref[...]
    g = lax.broadcasted_iota(jnp.int32, (G, 1), 0)
    m = (b == g).astype(jnp.float32)
    cnt_g = jnp.sum(m, axis=1, keepdims=True)
    cnode = jnp.sum(m * cnt_g, axis=0, keepdims=True)
    c = jnp.maximum(cnode, 1.0)
    sqrtcnt_ref[...] = jnp.sqrt(c)
    invcnt_ref[...] = 1.0 / c


_prep = pl.pallas_call(
    _prep_body,
    out_shape=(
        jax.ShapeDtypeStruct((1, NPAD), jnp.float32),
        jax.ShapeDtypeStruct((1, NPAD), jnp.float32),
    ),
)


def _sc_attract_body(cx_hbm, cy_hbm, sq_hbm, row_hbm, col_hbm, out_hbm,
                     cxv, cyv, sqv, rv, cv, fxv, fyv):
    wid = lax.axis_index("s") * 2 + lax.axis_index("c")
    base = wid * EPT
    pltpu.sync_copy(cx_hbm, cxv)
    pltpu.sync_copy(cy_hbm, cyv)
    pltpu.sync_copy(sq_hbm, sqv)
    pltpu.sync_copy(row_hbm.at[pl.ds(base, EPT)], rv)
    pltpu.sync_copy(col_hbm.at[pl.ds(base, EPT)], cv)

    zero16 = jnp.zeros((LANES,), jnp.float32)

    def _zero(i, carry):
        fxv[pl.ds(i * LANES, LANES)] = zero16
        fyv[pl.ds(i * LANES, LANES)] = zero16
        return carry

    lax.fori_loop(0, NPAD // LANES, _zero, 0)

    def _edges(i, carry):
        r = rv[pl.ds(i * LANES, LANES)]
        c = cv[pl.ds(i * LANES, LANES)]
        xr = plsc.load_gather(cxv, [r])
        yr = plsc.load_gather(cyv, [r])
        xc = plsc.load_gather(cxv, [c])
        yc = plsc.load_gather(cyv, [c])
        sq = plsc.load_gather(sqv, [r])
        dx = xr - xc
        dy = yr - yc
        d2 = dx * dx + dy * dy + 1e-20
        ib = plsc.bitcast(d2, jnp.int32)
        y = plsc.bitcast((ib >> 1) + jnp.int32(0x1FBD1DF5), jnp.float32)
        y = 0.5 * (y + d2 / y)
        y = 0.5 * (y + d2 / y)
        y = 0.5 * (y + d2 / y)
        coef = -((y + EPS) * sq)
        ax = coef * dx
        ay = coef * dy
        plsc.addupdate_scatter(fxv, [r], ax)
        plsc.addupdate_scatter(fyv, [r], ay)
        plsc.addupdate_scatter(fxv, [c], -ax)
        plsc.addupdate_scatter(fyv, [c], -ay)
        return carry

    lax.fori_loop(0, EPT // LANES, _edges, 0)

    pltpu.sync_copy(fxv, out_hbm.at[wid, 0])
    pltpu.sync_copy(fyv, out_hbm.at[wid, 1])


@functools.cache
def _sc_attract_kernel():
    mesh = plsc.VectorSubcoreMesh(core_axis_name="c", subcore_axis_name="s")
    return pl.kernel(
        _sc_attract_body,
        mesh=mesh,
        compiler_params=pltpu.CompilerParams(needs_layout_passes=False),
        out_type=jax.ShapeDtypeStruct((NTILES, 2, NPAD), jnp.float32),
        scratch_types=[
            pltpu.VMEM((NPAD,), jnp.float32),
            pltpu.VMEM((NPAD,), jnp.float32),
            pltpu.VMEM((NPAD,), jnp.float32),
            pltpu.VMEM((EPT,), jnp.int32),
            pltpu.VMEM((EPT,), jnp.int32),
            pltpu.VMEM((NPAD,), jnp.float32),
            pltpu.VMEM((NPAD,), jnp.float32),
        ],
    )


def _step_body(cx_ref, cy_ref, bf_ref, ic_ref, cxT_ref, cyT_ref, bfT_ref,
               par_ref, lo_ref, hi_ref, alpha_ref, ncx_ref, ncy_ref):
    i = pl.program_id(0)
    xi = cx_ref[...]
    yi = cy_ref[...]
    bi = bf_ref[...]
    ki2 = ic_ref[...]
    gi = B * i + lax.broadcasted_iota(jnp.int32, (1, B), 1)

    def jbody(j, carry):
        sx, sy = carry
        off = j * B
        xj = cxT_ref[pl.ds(off, B), :]
        yj = cyT_ref[pl.ds(off, B), :]
        bj = bfT_ref[pl.ds(off, B), :]
        gj = B * j + lax.broadcasted_iota(jnp.int32, (B, 1), 0)
        dx = xi - xj
        dy = yi - yj
        eye = (gi == gj).astype(jnp.float32)
        d2 = dx * dx + dy * dy + eye
        dist = jnp.sqrt(d2) + EPS
        w = jnp.where(bi == bj, 1.0 / (dist * dist), 0.0)
        sx = sx + jnp.sum(w * dx, axis=0, keepdims=True)
        sy = sy + jnp.sum(w * dy, axis=0, keepdims=True)
        return sx, sy

    z = jnp.zeros((1, B), jnp.float32)
    sx, sy = lax.fori_loop(lo_ref[i], hi_ref[i], jbody, (z, z))

    fax = jnp.sum(par_ref[:, 0, pl.ds(i * B, B)], axis=0, keepdims=True)
    fay = jnp.sum(par_ref[:, 1, pl.ds(i * B, B)], axis=0, keepdims=True)
    alpha = alpha_ref[0, 0]
    dxt = alpha * (fax + ki2 * sx)
    dyt = alpha * (fay + ki2 * sy)
    nrm = jnp.sqrt(dxt * dxt + dyt * dyt + 1e-20)
    scale = jnp.minimum(CLAMP_STEP / (nrm + 1e-9), 1.0)
    ncx_ref[...] = xi + dxt * scale
    ncy_ref[...] = yi + dyt * scale


_blk = pl.BlockSpec((1, B), lambda i: (0, i))
_colT = pl.BlockSpec((NPAD, 1), lambda i: (0, 0))
_smem = pl.BlockSpec(memory_space=pltpu.SMEM)

_step = pl.pallas_call(
    _step_body,
    grid=(T,),
    in_specs=[
        _blk,
        _blk,
        _blk,
        _blk,
        _colT,
        _colT,
        _colT,
        pl.BlockSpec((NTILES, 2, NPAD), lambda i: (0, 0, 0)),
        _smem,
        _smem,
        _smem,
    ],
    out_specs=(_blk, _blk),
    out_shape=(
        jax.ShapeDtypeStruct((1, NPAD), jnp.float32),
        jax.ShapeDtypeStruct((1, NPAD), jnp.float32),
    ),
)


def kernel(x, alpha, edge_index, batch):
    row = edge_index[0]
    col = edge_index[1]
    cx = jnp.pad(x[:, -2], (0, NPAD - N))
    cy = jnp.pad(x[:, -1], (0, NPAD - N))
    batch_p = jnp.pad(batch, (0, NPAD - N), constant_values=PAD_SENT)
    bf = batch_p.astype(jnp.float32)

    sqrtcnt, invcnt = _prep(batch_p.reshape(1, NPAD))

    tiles = bf.reshape(T, B)
    tmin = tiles[:, 0]
    tmax = tiles[:, -1]
    lo = jnp.searchsorted(tmax, tmin, side="left").astype(jnp.int32)
    hi = jnp.searchsorted(tmin, tmax, side="right").astype(jnp.int32)

    alpha_s = jnp.reshape(alpha, (1, 1)).astype(jnp.float32)
    sq1 = sqrtcnt.reshape(NPAD)
    bf2 = bf.reshape(1, NPAD)
    bfT = bf.reshape(NPAD, 1)

    cx2 = cx.reshape(1, NPAD)
    cy2 = cy.reshape(1, NPAD)
    for _ in range(STEPS):
        par = _sc_attract_kernel()(
            cx2.reshape(NPAD), cy2.reshape(NPAD), sq1, row, col)
        cx2, cy2 = _step(cx2, cy2, bf2, invcnt,
                         cx2.reshape(NPAD, 1), cy2.reshape(NPAD, 1), bfT,
                         par, lo, hi, alpha_s)
    return jnp.stack([cx2[0, :N], cy2[0, :N]], axis=1)


# speedup vs baseline: 66.7066x; 1.5834x over previous
import functools

import jax
import jax.numpy as jnp
from jax import lax
from jax.experimental import pallas as pl
from jax.experimental.pallas import tpu as pltpu
from jax.experimental.pallas import tpu_sc as plsc

N = 10000
E = 320000
G = 100
STEPS = 3
EPS = 0.01
CLAMP_STEP = 0.1

B = 256
NPAD = 10240
T = NPAD // B
NTILES = 32
EPT = E // NTILES
LANES = 16
UNROLL = 5
PAD_SENT = 2 ** 30


def _prep_body(batch_ref, sqrtcnt_ref, invcnt_ref):
    b = batch_ref[...]
    g = lax.broadcasted_iota(jnp.int32, (G, 1), 0)
    m = (b == g).astype(jnp.float32)
    cnt_g = jnp.sum(m, axis=1, keepdims=True)
    cnode = jnp.sum(m * cnt_g, axis=0, keepdims=True)
    c = jnp.maximum(cnode, 1.0)
    sqrtcnt_ref[...] = jnp.sqrt(c)
    invcnt_ref[...] = 1.0 / c


_prep = pl.pallas_call(
    _prep_body,
    out_shape=(
        jax.ShapeDtypeStruct((1, NPAD), jnp.float32),
        jax.ShapeDtypeStruct((1, NPAD), jnp.float32),
    ),
)


def _sc_attract_body(cx_hbm, cy_hbm, sq_hbm, row_hbm, col_hbm, out_hbm,
                     cxv, cyv, sqv, rv, cv, fxv, fyv):
    wid = lax.axis_index("s") * 2 + lax.axis_index("c")
    base = wid * EPT
    pltpu.sync_copy(cx_hbm, cxv)
    pltpu.sync_copy(cy_hbm, cyv)
    pltpu.sync_copy(sq_hbm, sqv)
    pltpu.sync_copy(row_hbm.at[pl.ds(base, EPT)], rv)
    pltpu.sync_copy(col_hbm.at[pl.ds(base, EPT)], cv)

    zero16 = jnp.zeros((LANES,), jnp.float32)

    def _zero(i, carry):
        fxv[pl.ds(i * LANES, LANES)] = zero16
        fyv[pl.ds(i * LANES, LANES)] = zero16
        return carry

    lax.fori_loop(0, NPAD // LANES, _zero, 0)

    def _edges(i, carry):
        for u in range(UNROLL):
            o = (i * UNROLL + u) * LANES
            r = rv[pl.ds(o, LANES)]
            c = cv[pl.ds(o, LANES)]
            xr = plsc.load_gather(cxv, [r])
            yr = plsc.load_gather(cyv, [r])
            xc = plsc.load_gather(cxv, [c])
            yc = plsc.load_gather(cyv, [c])
            sq = plsc.load_gather(sqv, [r])
            dx = xr - xc
            dy = yr - yc
            d2 = dx * dx + dy * dy + 1e-20
            ib = plsc.bitcast(d2, jnp.int32)
            y = plsc.bitcast(jnp.int32(0x5F3759DF) - (ib >> 1), jnp.float32)
            h = 0.5 * d2
            y = y * (1.5 - h * y * y)
            y = y * (1.5 - h * y * y)
            y = y * (1.5 - h * y * y)
            coef = -(d2 * y + EPS) * sq
            ax = coef * dx
            ay = coef * dy
            plsc.addupdate_scatter(fxv, [r], ax)
            plsc.addupdate_scatter(fyv, [r], ay)
            plsc.addupdate_scatter(fxv, [c], -ax)
            plsc.addupdate_scatter(fyv, [c], -ay)
        return carry

    lax.fori_loop(0, EPT // (LANES * UNROLL), _edges, 0)

    pltpu.sync_copy(fxv, out_hbm.at[0, wid])
    pltpu.sync_copy(fyv, out_hbm.at[1, wid])


@functools.cache
def _sc_attract_kernel():
    mesh = plsc.VectorSubcoreMesh(core_axis_name="c", subcore_axis_name="s")
    return pl.kernel(
        _sc_attract_body,
        mesh=mesh,
        compiler_params=pltpu.CompilerParams(needs_layout_passes=False),
        out_type=jax.ShapeDtypeStruct((2, NTILES, NPAD), jnp.float32),
        scratch_types=[
            pltpu.VMEM((NPAD,), jnp.float32),
            pltpu.VMEM((NPAD,), jnp.float32),
            pltpu.VMEM((NPAD,), jnp.float32),
            pltpu.VMEM((EPT,), jnp.int32),
            pltpu.VMEM((EPT,), jnp.int32),
            pltpu.VMEM((NPAD,), jnp.float32),
            pltpu.VMEM((NPAD,), jnp.float32),
        ],
    )


def _rep_body(co_ref, bf_ref, ic_ref, coT_ref, bfT_ref, lo_ref, hi_ref,
              rep_ref):
    i = pl.program_id(0)
    xi = co_ref[0:1, :]
    yi = co_ref[1:2, :]
    bi = bf_ref[...]
    ki2 = ic_ref[...]
    gi = B * i + lax.broadcasted_iota(jnp.int32, (1, B), 1)

    def jbody(j, carry):
        sx, sy = carry
        off = j * B
        xj = coT_ref[pl.ds(off, B), 0:1]
        yj = coT_ref[pl.ds(off, B), 1:2]
        bj = bfT_ref[pl.ds(off, B), :]
        gj = B * j + lax.broadcasted_iota(jnp.int32, (B, 1), 0)
        dx = xi - xj
        dy = yi - yj
        eye = (gi == gj).astype(jnp.float32)
        d2 = dx * dx + dy * dy + eye
        dist = jnp.sqrt(d2) + EPS
        w = jnp.where(bi == bj, 1.0 / (dist * dist), 0.0)
        sx = sx + jnp.sum(w * dx, axis=0, keepdims=True)
        sy = sy + jnp.sum(w * dy, axis=0, keepdims=True)
        return sx, sy

    z = jnp.zeros((1, B), jnp.float32)
    sx, sy = lax.fori_loop(lo_ref[i], hi_ref[i], jbody, (z, z))
    rep_ref[0:1, :] = ki2 * sx
    rep_ref[1:2, :] = ki2 * sy


_smem = pl.BlockSpec(memory_space=pltpu.SMEM)

_rep = pl.pallas_call(
    _rep_body,
    grid=(T,),
    in_specs=[
        pl.BlockSpec((2, B), lambda i: (0, i)),
        pl.BlockSpec((1, B), lambda i: (0, i)),
        pl.BlockSpec((1, B), lambda i: (0, i)),
        pl.BlockSpec((NPAD, 2), lambda i: (0, 0)),
        pl.BlockSpec((NPAD, 1), lambda i: (0, 0)),
        _smem,
        _smem,
    ],
    out_specs=pl.BlockSpec((2, B), lambda i: (0, i)),
    out_shape=jax.ShapeDtypeStruct((2, NPAD), jnp.float32),
)


def _upd_body(co_ref, rep_ref, par_ref, alpha_ref, out_ref):
    fax = jnp.sum(par_ref[0], axis=0, keepdims=True)
    fay = jnp.sum(par_ref[1], axis=0, keepdims=True)
    a = alpha_ref[0, 0]
    dxt = a * (fax + rep_ref[0:1, :])
    dyt = a * (fay + rep_ref[1:2, :])
    nrm = jnp.sqrt(dxt * dxt + dyt * dyt + 1e-20)
    scale = jnp.minimum(CLAMP_STEP / (nrm + 1e-9), 1.0)
    out_ref[0:1, :] = co_ref[0:1, :] + dxt * scale
    out_ref[1:2, :] = co_ref[1:2, :] + dyt * scale


_upd = pl.pallas_call(
    _upd_body,
    in_specs=[
        pl.BlockSpec((2, NPAD), lambda: (0, 0)),
        pl.BlockSpec((2, NPAD), lambda: (0, 0)),
        pl.BlockSpec((2, NTILES, NPAD), lambda: (0, 0, 0)),
        _smem,
    ],
    out_specs=pl.BlockSpec((2, NPAD), lambda: (0, 0)),
    out_shape=jax.ShapeDtypeStruct((2, NPAD), jnp.float32),
)


def kernel(x, alpha, edge_index, batch):
    row = edge_index[0]
    col = edge_index[1]
    coT = jnp.pad(x[:, -2:], ((0, NPAD - N), (0, 0)))
    co = coT.T
    batch_p = jnp.pad(batch, (0, NPAD - N), constant_values=PAD_SENT)
    bf = batch_p.astype(jnp.float32)

    sqrtcnt, invcnt = _prep(batch_p.reshape(1, NPAD))

    tiles = bf.reshape(T, B)
    tmin = tiles[:, 0]
    tmax = tiles[:, -1]
    lo = jnp.searchsorted(tmax, tmin, side="left").astype(jnp.int32)
    hi = jnp.searchsorted(tmin, tmax, side="right").astype(jnp.int32)

    alpha_s = jnp.reshape(alpha, (1, 1)).astype(jnp.float32)
    sq1 = sqrtcnt.reshape(NPAD)
    bf2 = bf.reshape(1, NPAD)
    bfT = bf.reshape(NPAD, 1)

    for step in range(STEPS):
        par = _sc_attract_kernel()(co[0], co[1], sq1, row, col)
        rep = _rep(co, bf2, invcnt, coT, bfT, lo, hi)
        co = _upd(co, rep, par, alpha_s)
        if step + 1 < STEPS:
            coT = co.T
    return co[:, :N].T
